# Initial kernel scaffold; baseline (speedup 1.0000x reference)
import jax
import jax.numpy as jnp
from jax.experimental import pallas as pl


def kernel(coords, vals, mask):
    raise NotImplementedError("write your pallas kernel here")


# speedup vs baseline: 1.6850x; 1.6850x over previous
import jax
import jax.numpy as jnp
from jax.experimental import pallas as pl
from jax.experimental.pallas import tpu as pltpu

_VMEM_LIMIT = 48 * 1024 * 1024


def _pool_body(vals_ref, mask_ref, out_ref):
    v = vals_ref[0]
    m = mask_ref[0].astype(jnp.float32)
    s = jnp.dot(m, v, preferred_element_type=jnp.float32)
    cnt = jnp.sum(m)
    inv = 1.0 / jnp.maximum(cnt, 1.0)
    out_ref[0] = s * inv


def kernel(coords, vals, mask):
    del coords
    bs, n, c = vals.shape
    mask3 = mask.reshape(bs, 1, n)

    cost = pl.CostEstimate(
        flops=2 * bs * n * c, transcendentals=0,
        bytes_accessed=bs * n * c * 4 + bs * n + bs * c * 4)

    out = pl.pallas_call(
        _pool_body,
        out_shape=jax.ShapeDtypeStruct((bs, 1, c), jnp.float32),
        grid=(bs,),
        in_specs=[
            pl.BlockSpec((1, n, c), lambda b: (b, 0, 0)),
            pl.BlockSpec((1, 1, n), lambda b: (b, 0, 0)),
        ],
        out_specs=pl.BlockSpec((1, 1, c), lambda b: (b, 0, 0)),
        compiler_params=pltpu.CompilerParams(
            dimension_semantics=("parallel",),
            vmem_limit_bytes=_VMEM_LIMIT),
        cost_estimate=cost,
    )(vals, mask3)
    return out.reshape(bs, c)
